# Initial kernel scaffold; baseline (speedup 1.0000x reference)
import jax
import jax.numpy as jnp
from jax.experimental import pallas as pl


def kernel(x_cat, tables, id_emb, type_emb):
    raise NotImplementedError("write your pallas kernel here")


# speedup vs baseline: 12.2440x; 12.2440x over previous
import functools

import jax
import jax.numpy as jnp
from jax import lax
from jax.experimental import pallas as pl
from jax.experimental.pallas import tpu as pltpu
from jax.experimental.pallas import tpu_sc as plsc

B = 16384
N_TOKENS = 43
VOCAB = 1000
DIM = 128

ROWS = B * N_TOKENS
NW = 32
CHUNK = 128
ROWS_PER_W = ROWS // NW
CHUNKS_PER_W = ROWS_PER_W // CHUNK


def _fuse_body(tab_ref, id_ref, type_ref, out_ref):
    out_ref[...] = tab_ref[...] + id_ref[...] + type_ref[...]


def _fuse_tables(tables, id_emb, type_emb):
    return pl.pallas_call(
        _fuse_body,
        grid=(N_TOKENS,),
        in_specs=[
            pl.BlockSpec((1, VOCAB, DIM), lambda t: (t, 0, 0)),
            pl.BlockSpec((1, 1, DIM), lambda t: (t, 0, 0)),
            pl.BlockSpec((1, 1, DIM), lambda t: (0, 0, 0)),
        ],
        out_specs=pl.BlockSpec((1, VOCAB, DIM), lambda t: (t, 0, 0)),
        out_shape=jax.ShapeDtypeStruct((N_TOKENS, VOCAB, DIM), jnp.float32),
    )(tables, id_emb.reshape(N_TOKENS, 1, DIM), type_emb)


def _idx_body(x_ref, out_ref):
    offs = lax.broadcasted_iota(jnp.int32, x_ref.shape, 1) * VOCAB
    out_ref[...] = x_ref[...] + offs


def _flat_indices(x_cat):
    blk = 2048
    return pl.pallas_call(
        _idx_body,
        grid=(B // blk,),
        in_specs=[pl.BlockSpec((blk, N_TOKENS), lambda i: (i, 0))],
        out_specs=pl.BlockSpec((blk, N_TOKENS), lambda i: (i, 0)),
        out_shape=jax.ShapeDtypeStruct((B, N_TOKENS), jnp.int32),
    )(x_cat)


def _gather_body(table_hbm, idx_hbm, out_hbm, idx_v, rows_v, sem):
    wid = lax.axis_index("s") * 2 + lax.axis_index("c")
    row_base = wid * ROWS_PER_W
    pltpu.sync_copy(idx_hbm.at[wid], idx_v)

    def step(j, carry):
        pltpu.async_copy(table_hbm.at[idx_v.at[j]], rows_v, sem).wait()
        pltpu.sync_copy(rows_v, out_hbm.at[pl.ds(row_base + j * CHUNK, CHUNK)])
        return carry

    lax.fori_loop(0, CHUNKS_PER_W, step, 0)


def _sc_gather(fused_table, idx2d):
    mesh = plsc.VectorSubcoreMesh(core_axis_name="c", subcore_axis_name="s")
    run = pl.kernel(
        _gather_body,
        out_type=jax.ShapeDtypeStruct((ROWS, DIM), jnp.float32),
        mesh=mesh,
        scratch_types=[
            pltpu.VMEM((CHUNKS_PER_W, CHUNK), jnp.int32),
            pltpu.VMEM((CHUNK, DIM), jnp.float32),
            pltpu.SemaphoreType.DMA,
        ],
    )
    return run(fused_table, idx2d)


@jax.jit
def kernel(x_cat, tables, id_emb, type_emb):
    fused = _fuse_tables(tables, id_emb, type_emb)
    fused = fused.reshape(N_TOKENS * VOCAB, DIM)
    idx = _flat_indices(x_cat.astype(jnp.int32))
    idx2d = idx.reshape(NW, CHUNKS_PER_W, CHUNK)
    out = _sc_gather(fused, idx2d)
    return out.reshape(B, N_TOKENS, DIM)


# speedup vs baseline: 13.9885x; 1.1425x over previous
import functools

import jax
import jax.numpy as jnp
from jax import lax
from jax.experimental import pallas as pl
from jax.experimental.pallas import tpu as pltpu
from jax.experimental.pallas import tpu_sc as plsc

B = 16384
N_TOKENS = 43
VOCAB = 1000
DIM = 128

ROWS = B * N_TOKENS
NW = 32
CHUNK = 128
ROWS_PER_W = ROWS // NW
CHUNKS_PER_W = ROWS_PER_W // CHUNK


def _fuse_body(tab_ref, id_ref, type_ref, out_ref):
    out_ref[...] = tab_ref[...] + id_ref[...] + type_ref[...]


def _fuse_tables(tables, id_emb, type_emb):
    return pl.pallas_call(
        _fuse_body,
        grid=(N_TOKENS,),
        in_specs=[
            pl.BlockSpec((1, VOCAB, DIM), lambda t: (t, 0, 0)),
            pl.BlockSpec((1, 1, DIM), lambda t: (t, 0, 0)),
            pl.BlockSpec((1, 1, DIM), lambda t: (0, 0, 0)),
        ],
        out_specs=pl.BlockSpec((1, VOCAB, DIM), lambda t: (t, 0, 0)),
        out_shape=jax.ShapeDtypeStruct((N_TOKENS, VOCAB, DIM), jnp.float32),
    )(tables, id_emb.reshape(N_TOKENS, 1, DIM), type_emb)


def _idx_body(x_ref, out_ref):
    offs = lax.broadcasted_iota(jnp.int32, x_ref.shape, 1) * VOCAB
    out_ref[...] = x_ref[...] + offs


def _flat_indices(x_cat):
    blk = 2048
    return pl.pallas_call(
        _idx_body,
        grid=(B // blk,),
        in_specs=[pl.BlockSpec((blk, N_TOKENS), lambda i: (i, 0))],
        out_specs=pl.BlockSpec((blk, N_TOKENS), lambda i: (i, 0)),
        out_shape=jax.ShapeDtypeStruct((B, N_TOKENS), jnp.int32),
    )(x_cat)


NBUF = 4
GROUPS = CHUNKS_PER_W // NBUF


def _gather_body(table_hbm, idx_hbm, out_hbm, idx_v,
                 b0, b1, b2, b3, g0, g1, g2, g3, o0, o1, o2, o3):
    bufs = (b0, b1, b2, b3)
    gsem = (g0, g1, g2, g3)
    osem = (o0, o1, o2, o3)
    wid = lax.axis_index("s") * 2 + lax.axis_index("c")
    row_base = wid * ROWS_PER_W
    pltpu.sync_copy(idx_hbm.at[wid], idx_v)

    for s in range(NBUF):
        pltpu.async_copy(table_hbm.at[idx_v.at[s]], bufs[s], gsem[s])

    def step(i, carry):
        for s in range(NBUF):
            j = i * NBUF + s
            pltpu.make_async_copy(
                table_hbm.at[idx_v.at[j]], bufs[s], gsem[s]).wait()
            pltpu.async_copy(
                bufs[s], out_hbm.at[pl.ds(row_base + j * CHUNK, CHUNK)],
                osem[s])
        for s in range(NBUF):
            j = i * NBUF + s
            jn = j + NBUF
            pltpu.make_async_copy(
                bufs[s], out_hbm.at[pl.ds(row_base + j * CHUNK, CHUNK)],
                osem[s]).wait()
            pltpu.async_copy(table_hbm.at[idx_v.at[jn]], bufs[s], gsem[s])
        return carry

    lax.fori_loop(0, GROUPS - 1, step, 0)

    for s in range(NBUF):
        j = (GROUPS - 1) * NBUF + s
        pltpu.make_async_copy(
            table_hbm.at[idx_v.at[j]], bufs[s], gsem[s]).wait()
        pltpu.async_copy(
            bufs[s], out_hbm.at[pl.ds(row_base + j * CHUNK, CHUNK)], osem[s])
    for s in range(NBUF):
        j = (GROUPS - 1) * NBUF + s
        pltpu.make_async_copy(
            bufs[s], out_hbm.at[pl.ds(row_base + j * CHUNK, CHUNK)],
            osem[s]).wait()


def _sc_gather(fused_table, idx2d):
    mesh = plsc.VectorSubcoreMesh(core_axis_name="c", subcore_axis_name="s")
    run = pl.kernel(
        _gather_body,
        out_type=jax.ShapeDtypeStruct((ROWS, DIM), jnp.float32),
        mesh=mesh,
        scratch_types=[
            pltpu.VMEM((CHUNKS_PER_W, CHUNK), jnp.int32),
        ] + [pltpu.VMEM((CHUNK, DIM), jnp.float32)] * NBUF
          + [pltpu.SemaphoreType.DMA] * (2 * NBUF),
    )
    return run(fused_table, idx2d)


@jax.jit
def kernel(x_cat, tables, id_emb, type_emb):
    fused = _fuse_tables(tables, id_emb, type_emb)
    fused = fused.reshape(N_TOKENS * VOCAB, DIM)
    idx = _flat_indices(x_cat.astype(jnp.int32))
    idx2d = idx.reshape(NW, CHUNKS_PER_W, CHUNK)
    out = _sc_gather(fused, idx2d)
    return out.reshape(B, N_TOKENS, DIM)


# speedup vs baseline: 14.2155x; 1.0162x over previous
import functools

import jax
import jax.numpy as jnp
from jax import lax
from jax.experimental import pallas as pl
from jax.experimental.pallas import tpu as pltpu
from jax.experimental.pallas import tpu_sc as plsc

B = 16384
N_TOKENS = 43
VOCAB = 1000
DIM = 128

ROWS = B * N_TOKENS
NW = 32
CHUNK = 128
ROWS_PER_W = ROWS // NW
CHUNKS_PER_W = ROWS_PER_W // CHUNK


FUSE_VBLK = 200


def _fuse_body(tab_ref, id_ref, type_ref, out_ref):
    out_ref[...] = tab_ref[...] + id_ref[...] + type_ref[...]


def _fuse_tables(tables, id_emb, type_emb):
    return pl.pallas_call(
        _fuse_body,
        grid=(VOCAB // FUSE_VBLK,),
        in_specs=[
            pl.BlockSpec((N_TOKENS, FUSE_VBLK, DIM), lambda v: (0, v, 0)),
            pl.BlockSpec((N_TOKENS, 1, DIM), lambda v: (0, 0, 0)),
            pl.BlockSpec((1, 1, DIM), lambda v: (0, 0, 0)),
        ],
        out_specs=pl.BlockSpec((N_TOKENS, FUSE_VBLK, DIM), lambda v: (0, v, 0)),
        out_shape=jax.ShapeDtypeStruct((N_TOKENS, VOCAB, DIM), jnp.float32),
    )(tables, id_emb.reshape(N_TOKENS, 1, DIM), type_emb)


def _idx_body(x_ref, out_ref):
    offs = lax.broadcasted_iota(jnp.int32, x_ref.shape, 1) * VOCAB
    out_ref[...] = x_ref[...] + offs


def _flat_indices(x_cat):
    blk = 2048
    return pl.pallas_call(
        _idx_body,
        grid=(B // blk,),
        in_specs=[pl.BlockSpec((blk, N_TOKENS), lambda i: (i, 0))],
        out_specs=pl.BlockSpec((blk, N_TOKENS), lambda i: (i, 0)),
        out_shape=jax.ShapeDtypeStruct((B, N_TOKENS), jnp.int32),
    )(x_cat)


NBUF = 4
GROUPS = CHUNKS_PER_W // NBUF


def _gather_body(table_hbm, idx_hbm, out_hbm, idx_v,
                 b0, b1, b2, b3, g0, g1, g2, g3, o0, o1, o2, o3):
    bufs = (b0, b1, b2, b3)
    gsem = (g0, g1, g2, g3)
    osem = (o0, o1, o2, o3)
    wid = lax.axis_index("s") * 2 + lax.axis_index("c")
    row_base = wid * ROWS_PER_W
    pltpu.sync_copy(idx_hbm.at[wid], idx_v)

    for s in range(NBUF):
        pltpu.async_copy(table_hbm.at[idx_v.at[s]], bufs[s], gsem[s])

    def step(i, carry):
        for s in range(NBUF):
            j = i * NBUF + s
            pltpu.make_async_copy(
                table_hbm.at[idx_v.at[j]], bufs[s], gsem[s]).wait()
            pltpu.async_copy(
                bufs[s], out_hbm.at[pl.ds(row_base + j * CHUNK, CHUNK)],
                osem[s])
        for s in range(NBUF):
            j = i * NBUF + s
            jn = j + NBUF
            pltpu.make_async_copy(
                bufs[s], out_hbm.at[pl.ds(row_base + j * CHUNK, CHUNK)],
                osem[s]).wait()
            pltpu.async_copy(table_hbm.at[idx_v.at[jn]], bufs[s], gsem[s])
        return carry

    lax.fori_loop(0, GROUPS - 1, step, 0)

    for s in range(NBUF):
        j = (GROUPS - 1) * NBUF + s
        pltpu.make_async_copy(
            table_hbm.at[idx_v.at[j]], bufs[s], gsem[s]).wait()
        pltpu.async_copy(
            bufs[s], out_hbm.at[pl.ds(row_base + j * CHUNK, CHUNK)], osem[s])
    for s in range(NBUF):
        j = (GROUPS - 1) * NBUF + s
        pltpu.make_async_copy(
            bufs[s], out_hbm.at[pl.ds(row_base + j * CHUNK, CHUNK)],
            osem[s]).wait()


def _sc_gather(fused_table, idx2d):
    mesh = plsc.VectorSubcoreMesh(core_axis_name="c", subcore_axis_name="s")
    run = pl.kernel(
        _gather_body,
        out_type=jax.ShapeDtypeStruct((ROWS, DIM), jnp.float32),
        mesh=mesh,
        scratch_types=[
            pltpu.VMEM((CHUNKS_PER_W, CHUNK), jnp.int32),
        ] + [pltpu.VMEM((CHUNK, DIM), jnp.float32)] * NBUF
          + [pltpu.SemaphoreType.DMA] * (2 * NBUF),
    )
    return run(fused_table, idx2d)


@jax.jit
def kernel(x_cat, tables, id_emb, type_emb):
    fused = _fuse_tables(tables, id_emb, type_emb)
    fused = fused.reshape(N_TOKENS * VOCAB, DIM)
    idx = _flat_indices(x_cat.astype(jnp.int32))
    idx2d = idx.reshape(NW, CHUNKS_PER_W, CHUNK)
    out = _sc_gather(fused, idx2d)
    return out.reshape(B, N_TOKENS, DIM)


# speedup vs baseline: 43.9550x; 3.0921x over previous
import functools

import jax
import jax.numpy as jnp
from jax import lax
from jax.experimental import pallas as pl
from jax.experimental.pallas import tpu as pltpu
from jax.experimental.pallas import tpu_sc as plsc

B = 16384
N_TOKENS = 43
VOCAB = 1000
DIM = 128

ROWS = B * N_TOKENS
NW = 32
CHUNK = 128
ROWS_PER_W = ROWS // NW
CHUNKS_PER_W = ROWS_PER_W // CHUNK


FUSE_VBLK = 200


def _fuse_body(tab_ref, id_ref, type_ref, out_ref):
    out_ref[...] = tab_ref[...] + id_ref[...] + type_ref[...]


def _fuse_tables(tables, id_emb, type_emb):
    return pl.pallas_call(
        _fuse_body,
        grid=(VOCAB // FUSE_VBLK,),
        in_specs=[
            pl.BlockSpec((N_TOKENS, FUSE_VBLK, DIM), lambda v: (0, v, 0)),
            pl.BlockSpec((N_TOKENS, 1, DIM), lambda v: (0, 0, 0)),
            pl.BlockSpec((1, 1, DIM), lambda v: (0, 0, 0)),
        ],
        out_specs=pl.BlockSpec((N_TOKENS, FUSE_VBLK, DIM), lambda v: (0, v, 0)),
        out_shape=jax.ShapeDtypeStruct((N_TOKENS, VOCAB, DIM), jnp.float32),
    )(tables, id_emb.reshape(N_TOKENS, 1, DIM), type_emb)


def _idx_body(x_ref, out_ref):
    offs = lax.broadcasted_iota(jnp.int32, x_ref.shape, 0) * VOCAB
    out_ref[...] = x_ref[...] + offs


def _flat_indices(x_cat_t):
    blk = 2048
    return pl.pallas_call(
        _idx_body,
        grid=(B // blk,),
        in_specs=[pl.BlockSpec((N_TOKENS, blk), lambda i: (0, i))],
        out_specs=pl.BlockSpec((N_TOKENS, blk), lambda i: (0, i)),
        out_shape=jax.ShapeDtypeStruct((N_TOKENS, B), jnp.int32),
    )(x_cat_t)


NBUF = 4
GROUPS = CHUNKS_PER_W // NBUF


def _gather_body(table_hbm, idx_hbm, out_hbm, idx_v,
                 b0, b1, b2, b3, g0, g1, g2, g3, o0, o1, o2, o3):
    bufs = (b0, b1, b2, b3)
    gsem = (g0, g1, g2, g3)
    osem = (o0, o1, o2, o3)
    wid = lax.axis_index("s") * 2 + lax.axis_index("c")
    row_base = wid * ROWS_PER_W
    pltpu.sync_copy(idx_hbm.at[wid], idx_v)

    for s in range(NBUF):
        pltpu.async_copy(table_hbm.at[idx_v.at[s]], bufs[s], gsem[s])

    def step(i, carry):
        for s in range(NBUF):
            j = i * NBUF + s
            pltpu.make_async_copy(
                table_hbm.at[idx_v.at[j]], bufs[s], gsem[s]).wait()
            pltpu.async_copy(
                bufs[s], out_hbm.at[pl.ds(row_base + j * CHUNK, CHUNK)],
                osem[s])
        for s in range(NBUF):
            j = i * NBUF + s
            jn = j + NBUF
            pltpu.make_async_copy(
                bufs[s], out_hbm.at[pl.ds(row_base + j * CHUNK, CHUNK)],
                osem[s]).wait()
            pltpu.async_copy(table_hbm.at[idx_v.at[jn]], bufs[s], gsem[s])
        return carry

    lax.fori_loop(0, GROUPS - 1, step, 0)

    for s in range(NBUF):
        j = (GROUPS - 1) * NBUF + s
        pltpu.make_async_copy(
            table_hbm.at[idx_v.at[j]], bufs[s], gsem[s]).wait()
        pltpu.async_copy(
            bufs[s], out_hbm.at[pl.ds(row_base + j * CHUNK, CHUNK)], osem[s])
    for s in range(NBUF):
        j = (GROUPS - 1) * NBUF + s
        pltpu.make_async_copy(
            bufs[s], out_hbm.at[pl.ds(row_base + j * CHUNK, CHUNK)],
            osem[s]).wait()


def _sc_gather(fused_table, idx2d):
    mesh = plsc.VectorSubcoreMesh(core_axis_name="c", subcore_axis_name="s")
    run = pl.kernel(
        _gather_body,
        out_type=jax.ShapeDtypeStruct((ROWS, DIM), jnp.float32),
        mesh=mesh,
        scratch_types=[
            pltpu.VMEM((CHUNKS_PER_W, CHUNK), jnp.int32),
        ] + [pltpu.VMEM((CHUNK, DIM), jnp.float32)] * NBUF
          + [pltpu.SemaphoreType.DMA] * (2 * NBUF),
    )
    return run(fused_table, idx2d)


@jax.jit
def kernel(x_cat, tables, id_emb, type_emb):
    fused = _fuse_tables(tables, id_emb, type_emb)
    fused = fused.reshape(N_TOKENS * VOCAB, DIM)
    idx = _flat_indices(x_cat.T.astype(jnp.int32))
    idx2d = idx.reshape(NW, CHUNKS_PER_W, CHUNK)
    out = _sc_gather(fused, idx2d)
    return out.reshape(N_TOKENS, B, DIM).transpose(1, 0, 2)


# speedup vs baseline: 44.2094x; 1.0058x over previous
import functools

import jax
import jax.numpy as jnp
from jax import lax
from jax.experimental import pallas as pl
from jax.experimental.pallas import tpu as pltpu
from jax.experimental.pallas import tpu_sc as plsc

B = 16384
N_TOKENS = 43
VOCAB = 1000
DIM = 128

ROWS = B * N_TOKENS
NW = 32
CHUNK = 128
ROWS_PER_W = ROWS // NW
CHUNKS_PER_W = ROWS_PER_W // CHUNK


FUSE_VBLK = 200


def _fuse_body(tab_ref, id_ref, type_ref, out_ref):
    out_ref[...] = tab_ref[...] + id_ref[...] + type_ref[...]


def _fuse_tables(tables, id_emb, type_emb):
    return pl.pallas_call(
        _fuse_body,
        grid=(VOCAB // FUSE_VBLK,),
        in_specs=[
            pl.BlockSpec((N_TOKENS, FUSE_VBLK, DIM), lambda v: (0, v, 0)),
            pl.BlockSpec((N_TOKENS, 1, DIM), lambda v: (0, 0, 0)),
            pl.BlockSpec((1, 1, DIM), lambda v: (0, 0, 0)),
        ],
        out_specs=pl.BlockSpec((N_TOKENS, FUSE_VBLK, DIM), lambda v: (0, v, 0)),
        out_shape=jax.ShapeDtypeStruct((N_TOKENS, VOCAB, DIM), jnp.float32),
    )(tables, id_emb.reshape(N_TOKENS, 1, DIM), type_emb)


def _idx_body(x_ref, out_ref):
    offs = lax.broadcasted_iota(jnp.int32, x_ref.shape, 0) * VOCAB
    out_ref[...] = x_ref[...] + offs


def _flat_indices(x_cat_t):
    return pl.pallas_call(
        _idx_body,
        out_shape=jax.ShapeDtypeStruct((N_TOKENS, B), jnp.int32),
    )(x_cat_t)


NBUF = 4
GROUPS = CHUNKS_PER_W // NBUF


def _gather_body(table_hbm, idx_hbm, out_hbm, idx_v, *scratch):
    bufs = scratch[:NBUF]
    gsem = scratch[NBUF:2 * NBUF]
    osem = scratch[2 * NBUF:]
    wid = lax.axis_index("s") * 2 + lax.axis_index("c")
    row_base = wid * ROWS_PER_W
    pltpu.sync_copy(idx_hbm.at[pl.ds(row_base, ROWS_PER_W)], idx_v)

    def idx_at(j):
        return idx_v.at[pl.ds(j * CHUNK, CHUNK)]

    for s in range(NBUF):
        pltpu.async_copy(table_hbm.at[idx_at(s)], bufs[s], gsem[s])

    def step(i, carry):
        for s in range(NBUF):
            j = i * NBUF + s
            pltpu.make_async_copy(
                table_hbm.at[idx_at(j)], bufs[s], gsem[s]).wait()
            pltpu.async_copy(
                bufs[s], out_hbm.at[pl.ds(row_base + j * CHUNK, CHUNK)],
                osem[s])
        for s in range(NBUF):
            j = i * NBUF + s
            jn = j + NBUF
            pltpu.make_async_copy(
                bufs[s], out_hbm.at[pl.ds(row_base + j * CHUNK, CHUNK)],
                osem[s]).wait()
            pltpu.async_copy(table_hbm.at[idx_at(jn)], bufs[s], gsem[s])
        return carry

    lax.fori_loop(0, GROUPS - 1, step, 0)

    for s in range(NBUF):
        j = (GROUPS - 1) * NBUF + s
        pltpu.make_async_copy(
            table_hbm.at[idx_at(j)], bufs[s], gsem[s]).wait()
        pltpu.async_copy(
            bufs[s], out_hbm.at[pl.ds(row_base + j * CHUNK, CHUNK)], osem[s])
    for s in range(NBUF):
        j = (GROUPS - 1) * NBUF + s
        pltpu.make_async_copy(
            bufs[s], out_hbm.at[pl.ds(row_base + j * CHUNK, CHUNK)],
            osem[s]).wait()


def _sc_gather(fused_table, idx_flat):
    mesh = plsc.VectorSubcoreMesh(core_axis_name="c", subcore_axis_name="s")
    run = pl.kernel(
        _gather_body,
        out_type=jax.ShapeDtypeStruct((ROWS, DIM), jnp.float32),
        mesh=mesh,
        scratch_types=[
            pltpu.VMEM((ROWS_PER_W,), jnp.int32),
        ] + [pltpu.VMEM((CHUNK, DIM), jnp.float32)] * NBUF
          + [pltpu.SemaphoreType.DMA] * (2 * NBUF),
    )
    return run(fused_table, idx_flat)


@jax.jit
def kernel(x_cat, tables, id_emb, type_emb):
    fused = _fuse_tables(tables, id_emb, type_emb)
    fused = fused.reshape(N_TOKENS * VOCAB, DIM)
    idx = _flat_indices(x_cat.T.astype(jnp.int32))
    out = _sc_gather(fused, idx.reshape(ROWS))
    return out.reshape(N_TOKENS, B, DIM).transpose(1, 0, 2)


# speedup vs baseline: 44.7629x; 1.0125x over previous
import functools

import jax
import jax.numpy as jnp
from jax import lax
from jax.experimental import pallas as pl
from jax.experimental.pallas import tpu as pltpu
from jax.experimental.pallas import tpu_sc as plsc

B = 16384
N_TOKENS = 43
VOCAB = 1000
DIM = 128

ROWS = B * N_TOKENS
NW = 32
CHUNK = 128
ROWS_PER_W = ROWS // NW
CHUNKS_PER_W = ROWS_PER_W // CHUNK


FUSE_VBLK = 200


def _fuse_body(tab_ref, id_ref, type_ref, out_ref):
    out_ref[...] = tab_ref[...] + id_ref[...] + type_ref[...]


def _fuse_tables(tables, id_emb, type_emb):
    return pl.pallas_call(
        _fuse_body,
        grid=(VOCAB // FUSE_VBLK,),
        in_specs=[
            pl.BlockSpec((N_TOKENS, FUSE_VBLK, DIM), lambda v: (0, v, 0)),
            pl.BlockSpec((N_TOKENS, 1, DIM), lambda v: (0, 0, 0)),
            pl.BlockSpec((1, 1, DIM), lambda v: (0, 0, 0)),
        ],
        out_specs=pl.BlockSpec((N_TOKENS, FUSE_VBLK, DIM), lambda v: (0, v, 0)),
        out_shape=jax.ShapeDtypeStruct((N_TOKENS, VOCAB, DIM), jnp.float32),
    )(tables, id_emb.reshape(N_TOKENS, 1, DIM), type_emb)


def _idx_body(x_ref, out_ref):
    offs = lax.broadcasted_iota(jnp.int32, x_ref.shape, 0) * VOCAB
    out_ref[...] = x_ref[...] + offs


def _flat_indices(x_cat_t):
    return pl.pallas_call(
        _idx_body,
        out_shape=jax.ShapeDtypeStruct((N_TOKENS, B), jnp.int32),
    )(x_cat_t)


NBUF = 4
GROUPS = CHUNKS_PER_W // NBUF


def _gather_body(table_hbm, idx_hbm, out_hbm, idx_v, *scratch):
    bufs = scratch[:NBUF]
    gsem = scratch[NBUF:2 * NBUF]
    osem = scratch[2 * NBUF:]
    wid = lax.axis_index("s") * 2 + lax.axis_index("c")
    row_base = wid * ROWS_PER_W
    pltpu.sync_copy(idx_hbm.at[pl.ds(row_base, ROWS_PER_W)], idx_v)

    def tbl_at(j):
        t = (row_base + j * CHUNK) // B
        return table_hbm.at[pl.ds(t * VOCAB, VOCAB)]

    def idx_at(j):
        return idx_v.at[pl.ds(j * CHUNK, CHUNK)]

    for s in range(NBUF):
        pltpu.async_copy(tbl_at(s).at[idx_at(s)], bufs[s], gsem[s])

    def step(i, carry):
        for s in range(NBUF):
            j = i * NBUF + s
            pltpu.make_async_copy(
                tbl_at(j).at[idx_at(j)], bufs[s], gsem[s]).wait()
            pltpu.async_copy(
                bufs[s], out_hbm.at[pl.ds(row_base + j * CHUNK, CHUNK)],
                osem[s])
        for s in range(NBUF):
            j = i * NBUF + s
            jn = j + NBUF
            pltpu.make_async_copy(
                bufs[s], out_hbm.at[pl.ds(row_base + j * CHUNK, CHUNK)],
                osem[s]).wait()
            pltpu.async_copy(tbl_at(jn).at[idx_at(jn)], bufs[s], gsem[s])
        return carry

    lax.fori_loop(0, GROUPS - 1, step, 0)

    for s in range(NBUF):
        j = (GROUPS - 1) * NBUF + s
        pltpu.make_async_copy(
            tbl_at(j).at[idx_at(j)], bufs[s], gsem[s]).wait()
        pltpu.async_copy(
            bufs[s], out_hbm.at[pl.ds(row_base + j * CHUNK, CHUNK)], osem[s])
    for s in range(NBUF):
        j = (GROUPS - 1) * NBUF + s
        pltpu.make_async_copy(
            bufs[s], out_hbm.at[pl.ds(row_base + j * CHUNK, CHUNK)],
            osem[s]).wait()


def _sc_gather(fused_table, idx_flat):
    mesh = plsc.VectorSubcoreMesh(core_axis_name="c", subcore_axis_name="s")
    run = pl.kernel(
        _gather_body,
        out_type=jax.ShapeDtypeStruct((ROWS, DIM), jnp.float32),
        mesh=mesh,
        scratch_types=[
            pltpu.VMEM((ROWS_PER_W,), jnp.int32),
        ] + [pltpu.VMEM((CHUNK, DIM), jnp.float32)] * NBUF
          + [pltpu.SemaphoreType.DMA] * (2 * NBUF),
    )
    return run(fused_table, idx_flat)


@jax.jit
def kernel(x_cat, tables, id_emb, type_emb):
    fused = _fuse_tables(tables, id_emb, type_emb)
    fused = fused.reshape(N_TOKENS * VOCAB, DIM)
    out = _sc_gather(fused, x_cat.T.reshape(ROWS))
    return out.reshape(N_TOKENS, B, DIM).transpose(1, 0, 2)
